# Initial kernel scaffold; baseline (speedup 1.0000x reference)
import jax
import jax.numpy as jnp
from jax.experimental import pallas as pl


def kernel(x, params):
    raise NotImplementedError("write your pallas kernel here")


# speedup vs baseline: 5.4340x; 5.4340x over previous
import functools

import jax
import jax.numpy as jnp
import numpy as np
from jax.experimental import pallas as pl
from jax.experimental.pallas import tpu as pltpu

_B = 2
_KP = 16
_DM = 128
_SQRT_DM = np.float32(np.sqrt(128.0))


def _rep_spec(shape):
    nd = len(shape)
    return pl.BlockSpec(shape, lambda *_: (0,) * nd)


def _batch_spec(shape):
    nd = len(shape)
    return pl.BlockSpec((None,) + shape, lambda b: (b,) + (0,) * nd)


def _fps_body(npoint, xyzR_ref, xyzT_ref, out_ref):
    n = xyzT_ref.shape[-1]
    x = xyzT_ref[0:1, :]
    y = xyzT_ref[1:2, :]
    z = xyzT_ref[2:3, :]
    lane = jax.lax.broadcasted_iota(jnp.int32, (1, n), 1)

    def body(i, carry):
        dist_min, far = carry
        out_ref[pl.ds(i, 1), :] = jnp.reshape(far, (1, 1))
        row = xyzR_ref[pl.ds(far, 1), :]
        cx = row[:, 0:1]
        cy = row[:, 1:2]
        cz = row[:, 2:3]
        dx = x - cx
        dy = y - cy
        dz = z - cz
        dist = dx * dx + dy * dy + dz * dz
        dist_min = jnp.minimum(dist_min, dist)
        m = jnp.max(dist_min)
        sel = jnp.where(dist_min == m, lane, n)
        far2 = jnp.min(sel)
        return dist_min, far2

    init = (jnp.full((1, n), 1e10, dtype=jnp.float32), jnp.int32(0))
    jax.lax.fori_loop(0, npoint, body, init)


def _fps(xyz, npoint):
    b, n, _ = xyz.shape
    xyz_t = jnp.transpose(xyz, (0, 2, 1))
    out = pl.pallas_call(
        functools.partial(_fps_body, npoint),
        grid=(b,),
        in_specs=[_batch_spec((n, 3)), _batch_spec((3, n))],
        out_specs=_batch_spec((npoint, 1)),
        out_shape=jax.ShapeDtypeStruct((b, npoint, 1), jnp.int32),
    )(xyz, xyz_t)
    return out


def _knn_body(k, has_qidx, *refs):
    if has_qidx:
        xyzR_ref, xyzT_ref, qidx_ref, out_ref = refs
    else:
        xyzR_ref, xyzT_ref, out_ref = refs
    n = xyzT_ref.shape[-1]
    if has_qidx:
        nq = qidx_ref.shape[0]
        lane_q = jax.lax.broadcasted_iota(jnp.int32, (nq, n), 1)
        oh = (lane_q == qidx_ref[:, :]).astype(jnp.float32)
        q = jnp.dot(oh, xyzR_ref[...], preferred_element_type=jnp.float32)
    else:
        nq = xyzR_ref.shape[0]
        q = xyzR_ref[...]
    qx = q[:, 0:1]
    qy = q[:, 1:2]
    qz = q[:, 2:3]
    dx = qx - xyzT_ref[0:1, :]
    dy = qy - xyzT_ref[1:2, :]
    dz = qz - xyzT_ref[2:3, :]
    d = dx * dx + dy * dy + dz * dz
    lane = jax.lax.broadcasted_iota(jnp.int32, (nq, n), 1)
    big = jnp.float32(np.inf)
    for j in range(k):
        m = jnp.min(d, axis=1, keepdims=True)
        sel = jnp.where(d == m, lane, n)
        amin = jnp.min(sel, axis=1, keepdims=True)
        out_ref[:, pl.ds(j, 1)] = amin
        d = jnp.where(lane == amin, big, d)


def _knn_self(xyz, k):
    b, n, _ = xyz.shape
    xyz_t = jnp.transpose(xyz, (0, 2, 1))
    return pl.pallas_call(
        functools.partial(_knn_body, k, False),
        grid=(b,),
        in_specs=[_batch_spec((n, 3)), _batch_spec((3, n))],
        out_specs=_batch_spec((n, k)),
        out_shape=jax.ShapeDtypeStruct((b, n, k), jnp.int32),
    )(xyz, xyz_t)


def _knn_fps(xyz, qidx, k):
    b, n, _ = xyz.shape
    nq = qidx.shape[1]
    xyz_t = jnp.transpose(xyz, (0, 2, 1))
    return pl.pallas_call(
        functools.partial(_knn_body, k, True),
        grid=(b,),
        in_specs=[_batch_spec((n, 3)), _batch_spec((3, n)),
                  _batch_spec((nq, 1))],
        out_specs=_batch_spec((nq, k)),
        out_shape=jax.ShapeDtypeStruct((b, nq, k), jnp.int32),
    )(xyz, xyz_t, qidx)


def _tb_pre_body(xyzR_ref, f_ref, fc1w_ref, fc1b_ref, wq_ref, wk_ref,
                 wv_ref, d1w_ref, q_ref, t_ref):
    x = jnp.dot(f_ref[...], fc1w_ref[...],
                preferred_element_type=jnp.float32) + fc1b_ref[...]
    q_ref[...] = jnp.dot(x, wq_ref[...], preferred_element_type=jnp.float32)
    t_ref[:, 0:128] = jnp.dot(x, wk_ref[...],
                              preferred_element_type=jnp.float32)
    t_ref[:, 128:256] = jnp.dot(x, wv_ref[...],
                                preferred_element_type=jnp.float32)
    t_ref[:, 256:384] = jnp.dot(xyzR_ref[...], d1w_ref[...],
                                preferred_element_type=jnp.float32)


def _tb_post_body(k, tile, d1b_ref, d2w_ref, d2b_ref, g1w_ref, g1b_ref,
                  g2w_ref, g2b_ref, fc2w_ref, fc2b_ref, t_ref, q_ref,
                  knn_ref, pre_ref, out_ref, a_sc, w_sc):
    n = t_ref.shape[0]
    tid = pl.program_id(1)
    pq = t_ref[pl.ds(tid * tile, tile), 256:384]
    qv = q_ref[...]
    knn = knn_ref[...]
    lane = jax.lax.broadcasted_iota(jnp.int32, (tile, n), 1)
    table = t_ref[...]
    for j in range(k):
        idx = knn[:, j:j + 1]
        oh = (lane == idx).astype(jnp.float32)
        g = jnp.dot(oh, table, preferred_element_type=jnp.float32)
        xk = g[:, 0:128]
        xv = g[:, 128:256]
        pg = g[:, 256:384]
        pos = jnp.maximum(pq - pg + d1b_ref[...], 0.0)
        pos = jnp.dot(pos, d2w_ref[...],
                      preferred_element_type=jnp.float32) + d2b_ref[...]
        u = qv - xk + pos
        a = jnp.maximum(jnp.dot(u, g1w_ref[...],
                                preferred_element_type=jnp.float32)
                        + g1b_ref[...], 0.0)
        a = jnp.dot(a, g2w_ref[...],
                    preferred_element_type=jnp.float32) + g2b_ref[...]
        a_sc[j] = a / _SQRT_DM
        w_sc[j] = xv + pos
    m = a_sc[0]
    for j in range(1, k):
        m = jnp.maximum(m, a_sc[j])
    s = jnp.zeros((tile, _DM), jnp.float32)
    acc = jnp.zeros((tile, _DM), jnp.float32)
    for j in range(k):
        e = jnp.exp(a_sc[j] - m)
        s = s + e
        acc = acc + e * w_sc[j]
    res = acc / s
    out_ref[...] = (jnp.dot(res, fc2w_ref[...],
                            preferred_element_type=jnp.float32)
                    + fc2b_ref[...] + pre_ref[...])


def _tb(p, xyz, feats, knn):
    b, n, d_in = feats.shape
    k = knn.shape[2]
    fc1w, fc1b = p['fc1']
    d1w, d1b = p['d1']
    d2w, d2b = p['d2']
    g1w, g1b = p['g1']
    g2w, g2b = p['g2']
    fc2w, fc2b = p['fc2']
    q, t = pl.pallas_call(
        _tb_pre_body,
        grid=(b,),
        in_specs=[_batch_spec((n, 3)), _batch_spec((n, d_in)),
                  _rep_spec(fc1w.shape), _rep_spec((1, _DM)),
                  _rep_spec(p['wq'].shape), _rep_spec(p['wk'].shape),
                  _rep_spec(p['wv'].shape), _rep_spec(d1w.shape)],
        out_specs=[_batch_spec((n, _DM)), _batch_spec((n, 384))],
        out_shape=[jax.ShapeDtypeStruct((b, n, _DM), jnp.float32),
                   jax.ShapeDtypeStruct((b, n, 384), jnp.float32)],
    )(xyz, feats, fc1w, fc1b.reshape(1, -1), p['wq'], p['wk'], p['wv'], d1w)

    tile = min(n, 256)
    nt = n // tile
    out = pl.pallas_call(
        functools.partial(_tb_post_body, k, tile),
        grid=(b, nt),
        in_specs=[_rep_spec((1, _DM)), _rep_spec(d2w.shape),
                  _rep_spec((1, _DM)), _rep_spec(g1w.shape),
                  _rep_spec((1, _DM)), _rep_spec(g2w.shape),
                  _rep_spec((1, _DM)), _rep_spec(fc2w.shape),
                  _rep_spec((1, d_in)),
                  pl.BlockSpec((None, n, 384), lambda b_, t_: (b_, 0, 0)),
                  pl.BlockSpec((None, tile, _DM), lambda b_, t_: (b_, t_, 0)),
                  pl.BlockSpec((None, tile, k), lambda b_, t_: (b_, t_, 0)),
                  pl.BlockSpec((None, tile, d_in), lambda b_, t_: (b_, t_, 0))],
        out_specs=pl.BlockSpec((None, tile, d_in), lambda b_, t_: (b_, t_, 0)),
        out_shape=jax.ShapeDtypeStruct((b, n, d_in), jnp.float32),
        scratch_shapes=[pltpu.VMEM((k, tile, _DM), jnp.float32),
                        pltpu.VMEM((k, tile, _DM), jnp.float32)],
    )(d1b.reshape(1, -1), d2w, d2b.reshape(1, -1), g1w, g1b.reshape(1, -1),
      g2w, g2b.reshape(1, -1), fc2w, fc2b.reshape(1, -1), t, q, knn, feats)
    return out


def _td_body(k, xyzR_ref, f_ref, fps_ref, knn_ref, l1wx_ref, l1wf_ref,
             l1b_ref, l2w_ref, l2b_ref, nxyz_ref, out_ref):
    n = xyzR_ref.shape[0]
    npt = fps_ref.shape[0]
    c_out = l2w_ref.shape[0]
    lane = jax.lax.broadcasted_iota(jnp.int32, (npt, n), 1)
    oh_fps = (lane == fps_ref[:, :]).astype(jnp.float32)
    new_xyz = jnp.dot(oh_fps, xyzR_ref[...],
                      preferred_element_type=jnp.float32)
    nxyz_ref[...] = new_xyz
    knn = knn_ref[...]
    m = jnp.full((npt, c_out), -jnp.inf, jnp.float32)
    for j in range(k):
        idx = knn[:, j:j + 1]
        oh = (lane == idx).astype(jnp.float32)
        gx = jnp.dot(oh, xyzR_ref[...],
                     preferred_element_type=jnp.float32) - new_xyz
        gf = jnp.dot(oh, f_ref[...], preferred_element_type=jnp.float32)
        h = (jnp.dot(gx, l1wx_ref[...], preferred_element_type=jnp.float32)
             + jnp.dot(gf, l1wf_ref[...], preferred_element_type=jnp.float32)
             + l1b_ref[...])
        h = jnp.maximum(h, 0.0)
        h = jnp.dot(h, l2w_ref[...],
                    preferred_element_type=jnp.float32) + l2b_ref[...]
        h = jnp.maximum(h, 0.0)
        m = jnp.maximum(m, h)
    out_ref[...] = m


def _td(p, xyz, feats, fps, knn):
    b, n, c_in = feats.shape
    npt = fps.shape[1]
    k = knn.shape[2]
    l1w, l1b = p['l1']
    l2w, l2b = p['l2']
    c_out = l2w.shape[1]
    nxyz, f_out = pl.pallas_call(
        functools.partial(_td_body, k),
        grid=(b,),
        in_specs=[_batch_spec((n, 3)), _batch_spec((n, c_in)),
                  _batch_spec((npt, 1)), _batch_spec((npt, k)),
                  _rep_spec((3, c_out)), _rep_spec((c_in, c_out)),
                  _rep_spec((1, c_out)), _rep_spec(l2w.shape),
                  _rep_spec((1, c_out))],
        out_specs=[_batch_spec((npt, 3)), _batch_spec((npt, c_out))],
        out_shape=[jax.ShapeDtypeStruct((b, npt, 3), jnp.float32),
                   jax.ShapeDtypeStruct((b, npt, c_out), jnp.float32)],
    )(xyz, feats, fps, knn, l1w[:3], l1w[3:], l1b.reshape(1, -1),
      l2w, l2b.reshape(1, -1))
    return nxyz, f_out


def _tu_body(fc_ref, xycR_ref, xycT_ref, ff_ref, xyf_ref, w1_ref, b1_ref,
             w2_ref, b2_ref, out_ref):
    nc = xycR_ref.shape[0]
    nf = xyf_ref.shape[0]
    f1 = jnp.maximum(jnp.dot(fc_ref[...], w1_ref[...],
                             preferred_element_type=jnp.float32)
                     + b1_ref[...], 0.0)
    f2 = jnp.maximum(jnp.dot(ff_ref[...], w2_ref[...],
                             preferred_element_type=jnp.float32)
                     + b2_ref[...], 0.0)
    dx = xyf_ref[:, 0:1] - xycT_ref[0:1, :]
    dy = xyf_ref[:, 1:2] - xycT_ref[1:2, :]
    dz = xyf_ref[:, 2:3] - xycT_ref[2:3, :]
    d = dx * dx + dy * dy + dz * dz
    lane = jax.lax.broadcasted_iota(jnp.int32, (nf, nc), 1)
    big = jnp.float32(np.inf)
    ws = []
    idxs = []
    for j in range(3):
        m = jnp.min(d, axis=1, keepdims=True)
        sel = jnp.where(d == m, lane, nc)
        amin = jnp.min(sel, axis=1, keepdims=True)
        ws.append(1.0 / jnp.maximum(m, 1e-10))
        idxs.append(amin)
        d = jnp.where(lane == amin, big, d)
    wsum = (ws[0] + ws[1]) + ws[2]
    acc = None
    for j in range(3):
        oh = (lane == idxs[j]).astype(jnp.float32)
        fj = jnp.dot(oh, f1, preferred_element_type=jnp.float32)
        term = (ws[j] / wsum) * fj
        acc = term if acc is None else acc + term
    out_ref[...] = acc + f2


def _tu(p, f_coarse, xyz_coarse, f_fine, xyz_fine):
    b, nc, _ = xyz_coarse.shape
    nf = xyz_fine.shape[1]
    w1, b1 = p['fc1']
    w2, b2 = p['fc2']
    d = w1.shape[1]
    xyc_t = jnp.transpose(xyz_coarse, (0, 2, 1))
    return pl.pallas_call(
        _tu_body,
        grid=(b,),
        in_specs=[_batch_spec(f_coarse.shape[1:]), _batch_spec((nc, 3)),
                  _batch_spec((3, nc)), _batch_spec(f_fine.shape[1:]),
                  _batch_spec((nf, 3)), _rep_spec(w1.shape),
                  _rep_spec((1, d)), _rep_spec(w2.shape), _rep_spec((1, d))],
        out_specs=_batch_spec((nf, d)),
        out_shape=jax.ShapeDtypeStruct((b, nf, d), jnp.float32),
    )(f_coarse, xyz_coarse, xyc_t, f_fine, xyz_fine, w1,
      b1.reshape(1, -1), w2, b2.reshape(1, -1))


def _mlp_body(relus, nlayer, *refs):
    x_ref = refs[0]
    out_ref = refs[-1]
    h = x_ref[...]
    for i in range(nlayer):
        w_ref = refs[1 + 2 * i]
        b_ref = refs[2 + 2 * i]
        h = jnp.dot(h, w_ref[...],
                    preferred_element_type=jnp.float32) + b_ref[...]
        if relus[i]:
            h = jnp.maximum(h, 0.0)
    out_ref[...] = h


def _mlp(x, layers, relus):
    b, n, _ = x.shape
    nlayer = len(layers)
    args = [x]
    specs = [_batch_spec(x.shape[1:])]
    for (w, bias) in layers:
        args.append(w)
        args.append(bias.reshape(1, -1))
        specs.append(_rep_spec(w.shape))
        specs.append(_rep_spec((1, w.shape[1])))
    d_out = layers[-1][0].shape[1]
    return pl.pallas_call(
        functools.partial(_mlp_body, relus, nlayer),
        grid=(b,),
        in_specs=specs,
        out_specs=_batch_spec((n, d_out)),
        out_shape=jax.ShapeDtypeStruct((b, n, d_out), jnp.float32),
    )(*args)


def kernel(x, params):
    xb = jnp.transpose(x, (0, 2, 1))
    xyz = xb
    f = _mlp(xb, [params['bb_fc1a'], params['bb_fc1b']], [True, False])
    f = _tb(params['bb_tb0'], xyz, f, _knn_self(xyz, _KP))
    fac = [(f, xyz)]
    npts = xyz.shape[1]
    for i in range(4):
        npts //= 4
        fps = _fps(xyz, npts)
        knn_d = _knn_fps(xyz, fps, _KP)
        xyz, f = _td(params['bb_td'][i], xyz, f, fps, knn_d)
        f = _tb(params['bb_tbs'][i], xyz, f, _knn_self(xyz, min(_KP, npts)))
        fac.append((f, xyz))
    feature, coord = fac[-1]
    h = _mlp(feature, [params['mlp2a'], params['mlp2b'], params['mlp2c']],
             [True, True, False])
    feature = _tb(params['t2'], coord, h,
                  _knn_self(coord, min(_KP, coord.shape[1])))
    for i in range(4):
        f_fine, c_fine = fac[-i - 2]
        feature = _tu(params['tu'][i], feature, coord, f_fine, c_fine)
        coord = c_fine
        feature = _tb(params['tbu'][i], coord, feature,
                      _knn_self(coord, min(_KP, coord.shape[1])))
    h = _mlp(feature, [params['mlp3a'], params['mlp3b'], params['mlp3c']],
             [True, True, False])
    return h
